# baseline (speedup 1.0000x reference)
import functools

import jax
import jax.numpy as jnp
from jax import lax
from jax.experimental import pallas as pl
from jax.experimental.pallas import tpu as pltpu, tpu_sc as plsc

_INFO = plsc.get_sparse_core_info()
_NC = _INFO.num_cores
_NS = _INFO.num_subcores
_NW = _NC * _NS
_L = _INFO.num_lanes

_ROWS, _COLS = 128, 32768
_SC_ROWS = 64
_TC_ROWS = _ROWS - _SC_ROWS
_CHUNK = _SC_ROWS * _COLS // _NW
_PIECE = 16384
_NPIECES = _CHUNK // _PIECE
_U = 8
_STEPS_PER_PIECE = _PIECE // _L
_I32MAX = 2**31 - 1

_TC_BLOCK_ROWS = 16
_TC_BLOCK_COLS = 16384
_TC_STEPS = _TC_ROWS // _TC_BLOCK_ROWS

_mesh = plsc.VectorSubcoreMesh(core_axis_name="c", subcore_axis_name="s")


@functools.partial(
    pl.kernel,
    mesh=_mesh,
    out_type=[
        jax.ShapeDtypeStruct((_NW, _L), jnp.float32),
        jax.ShapeDtypeStruct((_NW, _L), jnp.int32),
    ],
    scratch_types=[
        pltpu.VMEM((_PIECE,), jnp.float32),
        pltpu.VMEM((_PIECE,), jnp.float32),
        pltpu.VMEM((_PIECE,), jnp.float32),
        pltpu.VMEM((_L,), jnp.float32),
        pltpu.VMEM((_L,), jnp.int32),
        pltpu.SemaphoreType.DMA,
        pltpu.SemaphoreType.DMA,
        pltpu.SemaphoreType.DMA,
    ],
)
def _sc_scan(x_hbm, vals_hbm, idxs_hbm, buf0, buf1, buf2, st_v, st_i,
             sem0, sem1, sem2):
    wid = lax.axis_index("c") * _NS + lax.axis_index("s")
    base = wid * _CHUNK

    pieces_per_row = _COLS // _PIECE

    def piece_src(p):
        tp = wid * _NPIECES + p
        r = tp // pieces_per_row
        cb = (tp % pieces_per_row) * _PIECE
        cb = pl.multiple_of(cb, _PIECE)
        return x_hbm.at[r, pl.ds(cb, _PIECE)]

    bufs = (buf0, buf1, buf2)
    sems = (sem0, sem1, sem2)
    nbuf = len(bufs)
    cps = [
        pltpu.async_copy(piece_src(p), bufs[p % nbuf], sems[p % nbuf])
        for p in range(min(nbuf, _NPIECES))
    ]

    iota = lax.iota(jnp.int32, _L)
    bv = jnp.full((_L,), -jnp.inf, jnp.float32)
    bt = jnp.zeros((_L,), jnp.int32)

    for p in range(_NPIECES):
        buf = bufs[p % nbuf]
        cps[p % nbuf].wait()

        def body(i, carry, _p=p, _buf=buf):
            bv, bt = carry
            for u in range(_U):
                v = _buf[pl.ds((i * _U + u) * _L, _L)]
                s = _p * _STEPS_PER_PIECE + i * _U + u
                m = v > bv
                bv = jnp.maximum(bv, v)
                bt = jnp.where(m, s, bt)
            return bv, bt

        bv, bt = lax.fori_loop(0, _STEPS_PER_PIECE // _U, body, (bv, bt))

        if p + nbuf < _NPIECES:
            cps[p % nbuf] = pltpu.async_copy(
                piece_src(p + nbuf), buf, sems[p % nbuf]
            )

    st_v[...] = bv
    st_i[...] = bt * _L + iota + base
    pltpu.sync_copy(st_v, vals_hbm.at[wid])
    pltpu.sync_copy(st_i, idxs_hbm.at[wid])


def _tc_scan_body(x_ref, vals_ref, idxs_ref, mv, mi):
    j = pl.program_id(0)
    k = pl.program_id(1)

    @pl.when((j == 0) & (k == 0))
    def _():
        mv[...] = jnp.full((_TC_BLOCK_ROWS, 128), -jnp.inf, jnp.float32)
        mi[...] = jnp.zeros((_TC_BLOCK_ROWS, 128), jnp.int32)

    ibase = (
        lax.broadcasted_iota(jnp.int32, (_TC_BLOCK_ROWS, 128), 0) * _COLS
        + lax.broadcasted_iota(jnp.int32, (_TC_BLOCK_ROWS, 128), 1)
        + (_SC_ROWS * _COLS + j * (_TC_BLOCK_ROWS * _COLS) + k * _TC_BLOCK_COLS)
    )
    bv = mv[...]
    bi = mi[...]
    for sc in range(_TC_BLOCK_COLS // 128):
        sub = x_ref[:, pl.ds(sc * 128, 128)]
        m = sub > bv
        bv = jnp.maximum(bv, sub)
        bi = jnp.where(m, ibase + sc * 128, bi)
    mv[...] = bv
    mi[...] = bi

    @pl.when((j == _TC_STEPS - 1) & (k == _COLS // _TC_BLOCK_COLS - 1))
    def _():
        vals_ref[...] = bv
        idxs_ref[...] = bi


def _tc_scan(x):
    return pl.pallas_call(
        _tc_scan_body,
        grid=(_TC_STEPS, _COLS // _TC_BLOCK_COLS),
        in_specs=[
            pl.BlockSpec(
                (_TC_BLOCK_ROWS, _TC_BLOCK_COLS),
                lambda j, k: (j + _SC_ROWS // _TC_BLOCK_ROWS, k),
            )
        ],
        out_specs=[
            pl.BlockSpec((_TC_BLOCK_ROWS, 128), lambda j, k: (0, 0)),
            pl.BlockSpec((_TC_BLOCK_ROWS, 128), lambda j, k: (0, 0)),
        ],
        out_shape=[
            jax.ShapeDtypeStruct((_TC_BLOCK_ROWS, 128), jnp.float32),
            jax.ShapeDtypeStruct((_TC_BLOCK_ROWS, 128), jnp.int32),
        ],
        scratch_shapes=[
            pltpu.VMEM((_TC_BLOCK_ROWS, 128), jnp.float32),
            pltpu.VMEM((_TC_BLOCK_ROWS, 128), jnp.int32),
        ],
    )(x)


def _merge_body(scv_ref, sci_ref, tcv_ref, tci_ref, out_ref):
    scv = scv_ref[...]
    sci = sci_ref[...]
    tcv = tcv_ref[...]
    tci = tci_ref[...]
    mx = jnp.maximum(jnp.max(scv), jnp.max(tcv))
    c1 = jnp.min(jnp.where(scv == mx, sci, _I32MAX))
    c2 = jnp.min(jnp.where(tcv == mx, tci, _I32MAX))
    out_ref[...] = jnp.minimum(c1, c2).reshape(1, 1)


def _merge(scv, sci, tcv, tci):
    return pl.pallas_call(
        _merge_body,
        out_shape=jax.ShapeDtypeStruct((1, 1), jnp.int32),
    )(scv, sci, tcv, tci)


@jax.jit
def kernel(x):
    scv, sci = _sc_scan(x)
    tcv, tci = _tc_scan(x)
    out = _merge(scv, sci, tcv, tci)
    return out[0, 0].astype(jnp.int64)


# speedup vs baseline: 1.0223x; 1.0223x over previous
import functools

import jax
import jax.numpy as jnp
from jax import lax
from jax.experimental import pallas as pl
from jax.experimental.pallas import tpu as pltpu, tpu_sc as plsc

_INFO = plsc.get_sparse_core_info()
_NC = _INFO.num_cores
_NS = _INFO.num_subcores
_NW = _NC * _NS
_L = _INFO.num_lanes

_ROWS, _COLS = 128, 32768
_SC_ROWS = 48
_TC_ROWS = _ROWS - _SC_ROWS
_CHUNK = _SC_ROWS * _COLS // _NW
_PIECE = 16384
_NPIECES = _CHUNK // _PIECE
_U = 8
_STEPS_PER_PIECE = _PIECE // _L
_I32MAX = 2**31 - 1

_TC_BLOCK_ROWS = 16
_TC_BLOCK_COLS = 16384
_TC_STEPS = _TC_ROWS // _TC_BLOCK_ROWS

_mesh = plsc.VectorSubcoreMesh(core_axis_name="c", subcore_axis_name="s")


@functools.partial(
    pl.kernel,
    mesh=_mesh,
    out_type=[
        jax.ShapeDtypeStruct((_NW, _L), jnp.float32),
        jax.ShapeDtypeStruct((_NW, _L), jnp.int32),
    ],
    scratch_types=[
        pltpu.VMEM((_PIECE,), jnp.float32),
        pltpu.VMEM((_PIECE,), jnp.float32),
        pltpu.VMEM((_PIECE,), jnp.float32),
        pltpu.VMEM((_L,), jnp.float32),
        pltpu.VMEM((_L,), jnp.int32),
        pltpu.SemaphoreType.DMA,
        pltpu.SemaphoreType.DMA,
        pltpu.SemaphoreType.DMA,
    ],
)
def _sc_scan(x_hbm, vals_hbm, idxs_hbm, buf0, buf1, buf2, st_v, st_i,
             sem0, sem1, sem2):
    wid = lax.axis_index("c") * _NS + lax.axis_index("s")
    base = wid * _CHUNK

    pieces_per_row = _COLS // _PIECE

    def piece_src(p):
        tp = wid * _NPIECES + p
        r = tp // pieces_per_row
        cb = (tp % pieces_per_row) * _PIECE
        cb = pl.multiple_of(cb, _PIECE)
        return x_hbm.at[r, pl.ds(cb, _PIECE)]

    bufs = (buf0, buf1, buf2)
    sems = (sem0, sem1, sem2)
    nbuf = len(bufs)
    cps = [
        pltpu.async_copy(piece_src(p), bufs[p % nbuf], sems[p % nbuf])
        for p in range(min(nbuf, _NPIECES))
    ]

    iota = lax.iota(jnp.int32, _L)
    bv = jnp.full((_L,), -jnp.inf, jnp.float32)
    bt = jnp.zeros((_L,), jnp.int32)

    for p in range(_NPIECES):
        buf = bufs[p % nbuf]
        cps[p % nbuf].wait()

        def body(i, carry, _p=p, _buf=buf):
            bv, bt = carry
            for u in range(_U):
                v = _buf[pl.ds((i * _U + u) * _L, _L)]
                s = _p * _STEPS_PER_PIECE + i * _U + u
                m = v > bv
                bv = jnp.maximum(bv, v)
                bt = jnp.where(m, s, bt)
            return bv, bt

        bv, bt = lax.fori_loop(0, _STEPS_PER_PIECE // _U, body, (bv, bt))

        if p + nbuf < _NPIECES:
            cps[p % nbuf] = pltpu.async_copy(
                piece_src(p + nbuf), buf, sems[p % nbuf]
            )

    st_v[...] = bv
    st_i[...] = bt * _L + iota + base
    pltpu.sync_copy(st_v, vals_hbm.at[wid])
    pltpu.sync_copy(st_i, idxs_hbm.at[wid])


def _tc_scan_body(x_ref, vals_ref, idxs_ref, mv, mi):
    j = pl.program_id(0)
    k = pl.program_id(1)

    @pl.when((j == 0) & (k == 0))
    def _():
        mv[...] = jnp.full((_TC_BLOCK_ROWS, 128), -jnp.inf, jnp.float32)
        mi[...] = jnp.zeros((_TC_BLOCK_ROWS, 128), jnp.int32)

    ibase = (
        lax.broadcasted_iota(jnp.int32, (_TC_BLOCK_ROWS, 128), 0) * _COLS
        + lax.broadcasted_iota(jnp.int32, (_TC_BLOCK_ROWS, 128), 1)
        + (_SC_ROWS * _COLS + j * (_TC_BLOCK_ROWS * _COLS) + k * _TC_BLOCK_COLS)
    )
    bv = mv[...]
    bi = mi[...]
    for sc in range(_TC_BLOCK_COLS // 128):
        sub = x_ref[:, pl.ds(sc * 128, 128)]
        m = sub > bv
        bv = jnp.maximum(bv, sub)
        bi = jnp.where(m, ibase + sc * 128, bi)
    mv[...] = bv
    mi[...] = bi

    @pl.when((j == _TC_STEPS - 1) & (k == _COLS // _TC_BLOCK_COLS - 1))
    def _():
        vals_ref[...] = bv
        idxs_ref[...] = bi


def _tc_scan(x):
    return pl.pallas_call(
        _tc_scan_body,
        grid=(_TC_STEPS, _COLS // _TC_BLOCK_COLS),
        in_specs=[
            pl.BlockSpec(
                (_TC_BLOCK_ROWS, _TC_BLOCK_COLS),
                lambda j, k: (j + _SC_ROWS // _TC_BLOCK_ROWS, k),
            )
        ],
        out_specs=[
            pl.BlockSpec((_TC_BLOCK_ROWS, 128), lambda j, k: (0, 0)),
            pl.BlockSpec((_TC_BLOCK_ROWS, 128), lambda j, k: (0, 0)),
        ],
        out_shape=[
            jax.ShapeDtypeStruct((_TC_BLOCK_ROWS, 128), jnp.float32),
            jax.ShapeDtypeStruct((_TC_BLOCK_ROWS, 128), jnp.int32),
        ],
        scratch_shapes=[
            pltpu.VMEM((_TC_BLOCK_ROWS, 128), jnp.float32),
            pltpu.VMEM((_TC_BLOCK_ROWS, 128), jnp.int32),
        ],
    )(x)


def _merge_body(scv_ref, sci_ref, tcv_ref, tci_ref, out_ref):
    scv = scv_ref[...]
    sci = sci_ref[...]
    tcv = tcv_ref[...]
    tci = tci_ref[...]
    mx = jnp.maximum(jnp.max(scv), jnp.max(tcv))
    c1 = jnp.min(jnp.where(scv == mx, sci, _I32MAX))
    c2 = jnp.min(jnp.where(tcv == mx, tci, _I32MAX))
    out_ref[...] = jnp.minimum(c1, c2).reshape(1, 1)


def _merge(scv, sci, tcv, tci):
    return pl.pallas_call(
        _merge_body,
        out_shape=jax.ShapeDtypeStruct((1, 1), jnp.int32),
    )(scv, sci, tcv, tci)


@jax.jit
def kernel(x):
    scv, sci = _sc_scan(x)
    tcv, tci = _tc_scan(x)
    out = _merge(scv, sci, tcv, tci)
    return out[0, 0].astype(jnp.int64)
